# baseline (speedup 1.0000x reference)
import jax
import jax.numpy as jnp
from jax import lax
from jax.experimental import pallas as pl
from jax.experimental.pallas import tpu as pltpu


def _body(addval_ref, diff_ref, w_ref, o_ref):
    b, n = diff_ref.shape
    a = jnp.abs(diff_ref[...])
    idx = lax.broadcasted_iota(jnp.int32, (b, n), 1)
    mask = jnp.zeros((b, n), jnp.bool_)
    for _ in range(3):
        m = jnp.max(a, axis=1, keepdims=True)
        hit = a == m
        gi = jnp.min(jnp.where(hit, idx, n), axis=1, keepdims=True)
        sel = idx == gi
        mask = mask | sel
        a = jnp.where(sel, -1.0, a)
    o_ref[...] = w_ref[...] + jnp.where(mask, addval_ref[0], 0.0)


def kernel(difference, weight, epoch):
    b, n = difference.shape
    cond = (200 < epoch) & (epoch < 1000) & (epoch % 20 == 0)
    addval = jnp.where(cond, jnp.float32(1.0), jnp.float32(0.0)).reshape(1)
    return pl.pallas_call(
        _body,
        out_shape=jax.ShapeDtypeStruct((b, n), jnp.float32),
        in_specs=[
            pl.BlockSpec(memory_space=pltpu.SMEM),
            pl.BlockSpec((b, n), lambda: (0, 0)),
            pl.BlockSpec((b, n), lambda: (0, 0)),
        ],
        out_specs=pl.BlockSpec((b, n), lambda: (0, 0)),
    )(addval, difference, weight)
